# baseline (speedup 1.0000x reference)
import functools

import jax
import jax.numpy as jnp
import numpy as np
from jax import lax
from jax.experimental import pallas as pl
from jax.experimental.pallas import tpu as pltpu
from jax.experimental.pallas import tpu_sc as plsc

N_NODES = 10000
N_EDGES = 320000
D = 128
D_EDGE = 16
N_GRAPHS = 64
N_OUT = 64

NC = 2
NS = 16
EB = 40
EPW = N_EDGES // (NC * NS)
NIT = EPW // EB
NITC = 25
NCHUNK = NIT // NITC
ZROWS = 624
ZTAIL = N_NODES - NS * ZROWS


def _pack_select_mats():
    plo = np.zeros((D, D // 2), np.float32)
    phi = np.zeros((D, D // 2), np.float32)
    for j in range(D // 32):
        for k in range(16):
            plo[32 * j + k, 16 * j + k] = 1.0
            phi[32 * j + 16 + k, 16 * j + k] = 1.0
    return plo, phi


_PLO, _PHI = _pack_select_mats()


def _pack_bf16_pairs(y, plo, phi):
    ylo = jnp.dot(y, plo, preferred_element_type=jnp.float32)
    yhi = jnp.dot(y, phi, preferred_element_type=jnp.float32)

    def rne(v):
        u = lax.bitcast_convert_type(v, jnp.uint32)
        return (u + jnp.uint32(0x7FFF) + ((u >> 16) & jnp.uint32(1))) >> 16

    packed = rne(ylo) | (rne(yhi) << 16)
    return lax.bitcast_convert_type(packed, jnp.int32)


_EE_BLK = 2000


def _ee_body(ea_ref, w_ref, b_ref, plo_ref, phi_ref, out_ref):
    ee = (jnp.dot(ea_ref[...], w_ref[...], preferred_element_type=jnp.float32)
          + b_ref[...])
    out_ref[...] = _pack_bf16_pairs(ee, plo_ref[...], phi_ref[...])


def _edge_embed(edge_attr, w, b, plo, phi):
    return pl.pallas_call(
        _ee_body,
        grid=(N_EDGES // _EE_BLK,),
        in_specs=[
            pl.BlockSpec((_EE_BLK, D_EDGE), lambda i: (i, 0)),
            pl.BlockSpec((D_EDGE, D), lambda i: (0, 0)),
            pl.BlockSpec((1, D), lambda i: (0, 0)),
            pl.BlockSpec((D, D // 2), lambda i: (0, 0)),
            pl.BlockSpec((D, D // 2), lambda i: (0, 0)),
        ],
        out_specs=pl.BlockSpec((_EE_BLK, D // 2), lambda i: (i, 0)),
        out_shape=jax.ShapeDtypeStruct((N_EDGES, D // 2), jnp.int32),
    )(edge_attr, w, b.reshape(1, D), plo, phi)


def _xcast_body(x_ref, plo_ref, phi_ref, out_ref):
    out_ref[...] = _pack_bf16_pairs(x_ref[...], plo_ref[...], phi_ref[...])


def _xcast(x, plo, phi):
    return pl.pallas_call(
        _xcast_body,
        grid=(N_NODES // 1000,),
        in_specs=[
            pl.BlockSpec((1000, D), lambda i: (i, 0)),
            pl.BlockSpec((D, D // 2), lambda i: (0, 0)),
            pl.BlockSpec((D, D // 2), lambda i: (0, 0)),
        ],
        out_specs=pl.BlockSpec((1000, D // 2), lambda i: (i, 0)),
        out_shape=jax.ShapeDtypeStruct((N_NODES, D // 2), jnp.int32),
    )(x, plo, phi)


def _sc_edge_body(x_hbm, ee_hbm, src_hbm, dst_hbm, zeros_hbm, out_hbm,
                  src_v, dst_v, xg, eeb, mb, aggr, e_sem, g_sem, s_sem):
    c = lax.axis_index("c")
    s = lax.axis_index("s")

    pltpu.sync_copy(zeros_hbm, aggr.at[pl.ds(s * ZROWS, ZROWS)])

    @pl.when(s == NS - 1)
    def _():
        pltpu.sync_copy(zeros_hbm.at[pl.ds(0, ZTAIL)],
                        aggr.at[pl.ds(NS * ZROWS, ZTAIL)])

    w = c * NS + s

    def load_idx_chunk(ch):
        b = lax.rem(ch, 2)
        pltpu.sync_copy(src_hbm.at[w, ch], src_v.at[b])
        pltpu.sync_copy(dst_hbm.at[w, ch], dst_v.at[b])

    load_idx_chunk(0)
    plsc.subcore_barrier()

    ebase = w * EPW

    def src_row(i):
        return src_v.at[lax.rem(i // NITC, 2), lax.rem(i, NITC)]

    def dst_row(i):
        return dst_v.at[lax.rem(i // NITC, 2), lax.rem(i, NITC)]

    def start_ee(i, b):
        pltpu.async_copy(ee_hbm.at[pl.ds(ebase + i * EB, EB)], eeb.at[b], e_sem)

    def wait_ee(i, b):
        pltpu.make_async_copy(
            ee_hbm.at[pl.ds(ebase + i * EB, EB)], eeb.at[b], e_sem).wait()

    def start_gather(i, b):
        pltpu.async_copy(x_hbm.at[src_row(i)], xg.at[b], g_sem)

    def wait_gather(i, b):
        pltpu.make_async_copy(x_hbm.at[src_row(i)], xg.at[b], g_sem).wait()

    def start_scatter(i, b):
        pltpu.async_copy(mb.at[b], aggr.at[dst_row(i)], s_sem, add=True)

    def wait_scatter(i, b):
        pltpu.make_async_copy(mb.at[b], aggr.at[dst_row(i)], s_sem).wait()

    start_ee(0, 0)
    start_gather(0, 0)

    def edge_iter(i, carry):
        buf = lax.rem(i, 2)
        obuf = 1 - buf

        @pl.when(i >= 2)
        def _():
            wait_scatter(i - 2, buf)

        @pl.when(jnp.logical_and(i + 1 < NIT, lax.rem(i + 1, NITC) == 0))
        def _():
            load_idx_chunk((i + 1) // NITC)

        wait_ee(i, buf)
        wait_gather(i, buf)

        @pl.when(i + 1 < NIT)
        def _():
            start_ee(i + 1, obuf)
            start_gather(i + 1, obuf)

        hi_mask = jnp.int32(-65536)

        @plsc.parallel_loop(0, EB, unroll=4)
        def _(e):
            for j in range(D // 32):
                vx = xg[buf, e, pl.ds(j * 16, 16)]
                ve = eeb[buf, e, pl.ds(j * 16, 16)]
                xlo = lax.bitcast_convert_type(vx << 16, jnp.float32)
                elo = lax.bitcast_convert_type(ve << 16, jnp.float32)
                xhi = lax.bitcast_convert_type(vx & hi_mask, jnp.float32)
                ehi = lax.bitcast_convert_type(ve & hi_mask, jnp.float32)
                mb[buf, e, pl.ds(j * 32, 16)] = jnp.maximum(xlo + elo, 0.0)
                mb[buf, e, pl.ds(j * 32 + 16, 16)] = jnp.maximum(xhi + ehi, 0.0)

        start_scatter(i, buf)
        return carry

    lax.fori_loop(0, NIT, edge_iter, 0)
    wait_scatter(NIT - 2, lax.rem(NIT - 2, 2))
    wait_scatter(NIT - 1, lax.rem(NIT - 1, 2))
    plsc.subcore_barrier()

    pltpu.sync_copy(
        aggr.at[pl.ds(s * ZROWS, ZROWS)],
        out_hbm.at[c, pl.ds(s * ZROWS, ZROWS)],
    )

    @pl.when(s == NS - 1)
    def _():
        pltpu.sync_copy(aggr.at[pl.ds(NS * ZROWS, ZTAIL)],
                        out_hbm.at[c, pl.ds(NS * ZROWS, ZTAIL)])


@functools.cache
def _make_sc_edge():
    mesh = plsc.VectorSubcoreMesh(
        core_axis_name="c", subcore_axis_name="s", num_cores=NC, num_subcores=NS
    )
    return pl.kernel(
        _sc_edge_body,
        mesh=mesh,
        compiler_params=pltpu.CompilerParams(use_tc_tiling_on_sc=False),
        out_type=jax.ShapeDtypeStruct((NC, N_NODES, D), jnp.float32),
        scratch_types=[
            pltpu.VMEM((2, NITC, EB), jnp.int32),
            pltpu.VMEM((2, NITC, EB), jnp.int32),
            pltpu.VMEM((2, EB, D // 2), jnp.int32),
            pltpu.VMEM((2, EB, D // 2), jnp.int32),
            pltpu.VMEM((2, EB, D), jnp.float32),
            pltpu.VMEM_SHARED((N_NODES, D), jnp.float32),
            pltpu.SemaphoreType.DMA,
            pltpu.SemaphoreType.DMA,
            pltpu.SemaphoreType.DMA,
        ],
    )


def _sc_edge(h, ee, src, dst, zeros):
    src_r = src.reshape(NC * NS, NCHUNK, NITC, EB)
    dst_r = dst.reshape(NC * NS, NCHUNK, NITC, EB)
    return _make_sc_edge()(h, ee, src_r, dst_r, zeros)


_N_BLK = 1000
_N_GRID = N_NODES // _N_BLK


def _mlpA_body(x_ref, a0_ref, a1_ref, w_ref, b_ref, y_ref, st_ref):
    i = pl.program_id(0)
    h = x_ref[...] + a0_ref[...] + a1_ref[...]
    y = jnp.dot(h, w_ref[...], preferred_element_type=jnp.float32) + b_ref[...]
    y_ref[...] = y

    @pl.when(i == 0)
    def _():
        st_ref[...] = jnp.zeros_like(st_ref)

    st_ref[0:1, :] += jnp.sum(y, axis=0, keepdims=True)
    st_ref[1:2, :] += jnp.sum(y * y, axis=0, keepdims=True)


def _mlpA(x, a0, a1, w1, b1):
    return pl.pallas_call(
        _mlpA_body,
        grid=(_N_GRID,),
        in_specs=[
            pl.BlockSpec((_N_BLK, D), lambda i: (i, 0)),
            pl.BlockSpec((_N_BLK, D), lambda i: (i, 0)),
            pl.BlockSpec((_N_BLK, D), lambda i: (i, 0)),
            pl.BlockSpec((D, D), lambda i: (0, 0)),
            pl.BlockSpec((1, D), lambda i: (0, 0)),
        ],
        out_specs=[
            pl.BlockSpec((_N_BLK, D), lambda i: (i, 0)),
            pl.BlockSpec((8, D), lambda i: (0, 0)),
        ],
        out_shape=[
            jax.ShapeDtypeStruct((N_NODES, D), jnp.float32),
            jax.ShapeDtypeStruct((8, D), jnp.float32),
        ],
    )(x, a0, a1, w1, b1.reshape(1, D))


def _mlpB_body(y_ref, st_ref, g_ref, bb_ref, w2_ref, b2_ref, plo_ref, phi_ref,
               h_ref, hp_ref):
    inv_n = 1.0 / N_NODES
    mean = st_ref[0:1, :] * inv_n
    var = st_ref[1:2, :] * inv_n - mean * mean
    scale = lax.rsqrt(var + 1e-5) * g_ref[...]
    t = (y_ref[...] - mean) * scale + bb_ref[...]
    t = jnp.maximum(t, 0.0)
    h = jnp.dot(t, w2_ref[...], preferred_element_type=jnp.float32) + b2_ref[...]
    h = jnp.maximum(h, 0.0)
    h_ref[...] = h
    hp_ref[...] = _pack_bf16_pairs(h, plo_ref[...], phi_ref[...])


def _mlpB(y, st, g, bb, w2, b2, plo, phi):
    return pl.pallas_call(
        _mlpB_body,
        grid=(_N_GRID,),
        in_specs=[
            pl.BlockSpec((_N_BLK, D), lambda i: (i, 0)),
            pl.BlockSpec((8, D), lambda i: (0, 0)),
            pl.BlockSpec((1, D), lambda i: (0, 0)),
            pl.BlockSpec((1, D), lambda i: (0, 0)),
            pl.BlockSpec((D, D), lambda i: (0, 0)),
            pl.BlockSpec((1, D), lambda i: (0, 0)),
            pl.BlockSpec((D, D // 2), lambda i: (0, 0)),
            pl.BlockSpec((D, D // 2), lambda i: (0, 0)),
        ],
        out_specs=[
            pl.BlockSpec((_N_BLK, D), lambda i: (i, 0)),
            pl.BlockSpec((_N_BLK, D // 2), lambda i: (i, 0)),
        ],
        out_shape=[
            jax.ShapeDtypeStruct((N_NODES, D), jnp.float32),
            jax.ShapeDtypeStruct((N_NODES, D // 2), jnp.int32),
        ],
    )(y, st, g.reshape(1, D), bb.reshape(1, D), w2, b2.reshape(1, D), plo, phi)


def _pool_body(bi_ref, h1_ref, h2_ref, h3_ref, h4_ref,
               g1_ref, g2_ref, g3_ref, g4_ref):
    i = pl.program_id(0)

    @pl.when(i == 0)
    def _():
        g1_ref[...] = jnp.zeros_like(g1_ref)
        g2_ref[...] = jnp.zeros_like(g2_ref)
        g3_ref[...] = jnp.zeros_like(g3_ref)
        g4_ref[...] = jnp.zeros_like(g4_ref)

    seg = bi_ref[0, :, :]
    gids = lax.broadcasted_iota(jnp.int32, (N_GRAPHS, _N_BLK), 0)
    onehot = (gids == seg).astype(jnp.float32)
    g1_ref[...] += jnp.dot(onehot, h1_ref[...], preferred_element_type=jnp.float32)
    g2_ref[...] += jnp.dot(onehot, h2_ref[...], preferred_element_type=jnp.float32)
    g3_ref[...] += jnp.dot(onehot, h3_ref[...], preferred_element_type=jnp.float32)
    g4_ref[...] += jnp.dot(onehot, h4_ref[...], preferred_element_type=jnp.float32)


def _pool(batch_index, h1, h2, h3, h4):
    bi = batch_index.reshape(_N_GRID, 1, _N_BLK)
    gspec = pl.BlockSpec((N_GRAPHS, D), lambda i: (0, 0))
    hspec = pl.BlockSpec((_N_BLK, D), lambda i: (i, 0))
    return pl.pallas_call(
        _pool_body,
        grid=(_N_GRID,),
        in_specs=[pl.BlockSpec((1, 1, _N_BLK), lambda i: (i, 0, 0)),
                  hspec, hspec, hspec, hspec],
        out_specs=[gspec, gspec, gspec, gspec],
        out_shape=[jax.ShapeDtypeStruct((N_GRAPHS, D), jnp.float32)] * 4,
    )(bi, h1, h2, h3, h4)


def _head_body(g_ref, w1_ref, b1_ref, w2_ref, b2_ref, o_ref):
    t = jnp.dot(g_ref[...], w1_ref[...], preferred_element_type=jnp.float32) + b1_ref[...]
    t = jnp.maximum(t, 0.0)
    o = jnp.dot(t, w2_ref[...], preferred_element_type=jnp.float32) + b2_ref[...]
    o_ref[...] = jnp.log1p(jnp.exp(-jnp.abs(o))) + jnp.maximum(o, 0.0)


def _head(g, w1, b1, w2, b2):
    return pl.pallas_call(
        _head_body,
        out_shape=jax.ShapeDtypeStruct((N_GRAPHS, N_OUT), jnp.float32),
    )(g, w1, b1.reshape(1, 4 * D), w2, b2.reshape(1, N_OUT))


def kernel(x, graph_level_feats, edge_attr, edge_index, batch_index,
           lin_e1_w, lin_e1_b, mlp1_w1, mlp1_b1, bn1_g, bn1_b, mlp1_w2, mlp1_b2,
           lin_e2_w, lin_e2_b, mlp2_w1, mlp2_b1, bn2_g, bn2_b, mlp2_w2, mlp2_b2,
           lin_e3_w, lin_e3_b, mlp3_w1, mlp3_b1, bn3_g, bn3_b, mlp3_w2, mlp3_b2,
           lin_e4_w, lin_e4_b, mlp4_w1, mlp4_b1, bn4_g, bn4_b, mlp4_w2, mlp4_b2,
           lin1_w, lin1_b, lin2_w, lin2_b):
    src = edge_index[0]
    dst = edge_index[1]
    zeros = jnp.zeros((ZROWS, D), jnp.float32)
    plo = jnp.asarray(_PLO)
    phi = jnp.asarray(_PHI)

    def conv(h, hp, ee_w, ee_b, w1, b1, g, bb, w2, b2):
        ee = _edge_embed(edge_attr, ee_w, ee_b, plo, phi)
        agg = _sc_edge(hp, ee, src, dst, zeros)
        y, st = _mlpA(h, agg[0], agg[1], w1, b1)
        return _mlpB(y, st, g, bb, w2, b2, plo, phi)

    xp = _xcast(x, plo, phi)
    h1, h1p = conv(x, xp, lin_e1_w, lin_e1_b, mlp1_w1, mlp1_b1, bn1_g, bn1_b, mlp1_w2, mlp1_b2)
    h2, h2p = conv(h1, h1p, lin_e2_w, lin_e2_b, mlp2_w1, mlp2_b1, bn2_g, bn2_b, mlp2_w2, mlp2_b2)
    h3, _ = conv(h2, h2p, lin_e3_w, lin_e3_b, mlp3_w1, mlp3_b1, bn3_g, bn3_b, mlp3_w2, mlp3_b2)
    h4, _ = conv(h2, h2p, lin_e4_w, lin_e4_b, mlp4_w1, mlp4_b1, bn4_g, bn4_b, mlp4_w2, mlp4_b2)

    g1, g2, g3, g4 = _pool(batch_index, h1, h2, h3, h4)
    g = jnp.concatenate((g1, g2, g3, g4), axis=1)
    return _head(g, lin1_w, lin1_b, lin2_w, lin2_b)


# speedup vs baseline: 1.1395x; 1.1395x over previous
import functools

import jax
import jax.numpy as jnp
import numpy as np
from jax import lax
from jax.experimental import pallas as pl
from jax.experimental.pallas import tpu as pltpu
from jax.experimental.pallas import tpu_sc as plsc

N_NODES = 10000
N_EDGES = 320000
D = 128
D_EDGE = 16
N_GRAPHS = 64
N_OUT = 64

NC = 2
NS = 16
EBP = 40
EB = 2 * EBP
EPW = N_EDGES // (NC * NS)
NIT = EPW // EB
NITC = 5
NCHUNK = NIT // NITC
EPWP = EPW // 2
ZROWS = 624
ZTAIL = N_NODES - NS * ZROWS


def _pack_bf16_pairs(ylo, yhi):
    def rne(v):
        u = lax.bitcast_convert_type(v, jnp.uint32)
        return (u + jnp.uint32(0x7FFF) + ((u >> 16) & jnp.uint32(1))) >> 16

    packed = rne(ylo) | (rne(yhi) << 16)
    return lax.bitcast_convert_type(packed, jnp.int32)


_EE_BLK = 2000


_EE_GRID = (N_EDGES // 2) // _EE_BLK


def _ee_body(ea_lo_ref, ea_hi_ref, w_ref, b_ref, out_ref):
    lo = (jnp.dot(ea_lo_ref[...], w_ref[...], preferred_element_type=jnp.float32)
          + b_ref[...])
    hi = (jnp.dot(ea_hi_ref[...], w_ref[...], preferred_element_type=jnp.float32)
          + b_ref[...])
    out_ref[...] = _pack_bf16_pairs(lo, hi)


def _edge_embed(edge_attr, w, b):
    return pl.pallas_call(
        _ee_body,
        grid=(_EE_GRID,),
        in_specs=[
            pl.BlockSpec((_EE_BLK, D_EDGE), lambda i: (i, 0)),
            pl.BlockSpec((_EE_BLK, D_EDGE), lambda i: (i + _EE_GRID, 0)),
            pl.BlockSpec((D_EDGE, D), lambda i: (0, 0)),
            pl.BlockSpec((1, D), lambda i: (0, 0)),
        ],
        out_specs=pl.BlockSpec((_EE_BLK, D), lambda i: (i, 0)),
        out_shape=jax.ShapeDtypeStruct((N_EDGES // 2, D), jnp.int32),
    )(edge_attr, edge_attr, w, b.reshape(1, D))


def _sc_edge_body(x_hbm, ee_hbm, src_hbm, dst_hbm, zeros_hbm, out_hbm,
                  src_v, dst_v, xg, eeb, aggr, e_sem, g_sem, s_sem):
    c = lax.axis_index("c")
    s = lax.axis_index("s")

    pltpu.sync_copy(zeros_hbm, aggr.at[pl.ds(s * ZROWS, ZROWS)])

    @pl.when(s == NS - 1)
    def _():
        pltpu.sync_copy(zeros_hbm.at[pl.ds(0, ZTAIL)],
                        aggr.at[pl.ds(NS * ZROWS, ZTAIL)])

    w = c * NS + s

    def load_idx_chunk(ch):
        b = lax.rem(ch, 2)
        pltpu.sync_copy(src_hbm.at[w, ch], src_v.at[b])
        pltpu.sync_copy(dst_hbm.at[w, ch], dst_v.at[b])

    load_idx_chunk(0)
    plsc.subcore_barrier()

    pbase = w * EPWP

    def src_row(i):
        return src_v.at[lax.rem(i // NITC, 2), lax.rem(i, NITC)]

    def dst_row(i):
        return dst_v.at[lax.rem(i // NITC, 2), lax.rem(i, NITC)]

    def start_ee(i, b):
        pltpu.async_copy(
            ee_hbm.at[pl.ds(pbase + i * EBP, EBP)], eeb.at[b], e_sem)

    def wait_ee(i, b):
        pltpu.make_async_copy(
            ee_hbm.at[pl.ds(pbase + i * EBP, EBP)], eeb.at[b], e_sem).wait()

    def start_gather(i, b):
        pltpu.async_copy(x_hbm.at[src_row(i)], xg.at[b], g_sem)

    def wait_gather(i, b):
        pltpu.make_async_copy(x_hbm.at[src_row(i)], xg.at[b], g_sem).wait()

    def start_scatter(i, b):
        pltpu.async_copy(xg.at[b], aggr.at[dst_row(i)], s_sem, add=True)

    def wait_scatter(i, b):
        pltpu.make_async_copy(xg.at[b], aggr.at[dst_row(i)], s_sem).wait()

    start_ee(0, 0)
    start_gather(0, 0)

    hi_mask = jnp.int32(-65536)

    def edge_iter(i, carry):
        buf = lax.rem(i, 2)
        obuf = 1 - buf

        @pl.when(jnp.logical_and(i + 1 < NIT, lax.rem(i + 1, NITC) == 0))
        def _():
            load_idx_chunk((i + 1) // NITC)

        wait_ee(i, buf)
        wait_gather(i, buf)

        @plsc.parallel_loop(0, EBP, unroll=4)
        def _(t):
            for j in range(D // 16):
                cs = pl.ds(j * 16, 16)
                v = eeb[buf, t, cs]
                lo = lax.bitcast_convert_type(v << 16, jnp.float32)
                hi = lax.bitcast_convert_type(v & hi_mask, jnp.float32)
                xg[buf, 2 * t, cs] = jnp.maximum(xg[buf, 2 * t, cs] + lo, 0.0)
                xg[buf, 2 * t + 1, cs] = jnp.maximum(
                    xg[buf, 2 * t + 1, cs] + hi, 0.0)

        start_scatter(i, buf)

        @pl.when(i >= 1)
        def _():
            wait_scatter(i - 1, obuf)

        @pl.when(i + 1 < NIT)
        def _():
            start_ee(i + 1, obuf)
            start_gather(i + 1, obuf)

        return carry

    lax.fori_loop(0, NIT, edge_iter, 0)
    wait_scatter(NIT - 1, lax.rem(NIT - 1, 2))
    plsc.subcore_barrier()

    pltpu.sync_copy(
        aggr.at[pl.ds(s * ZROWS, ZROWS)],
        out_hbm.at[c, pl.ds(s * ZROWS, ZROWS)],
    )

    @pl.when(s == NS - 1)
    def _():
        pltpu.sync_copy(aggr.at[pl.ds(NS * ZROWS, ZTAIL)],
                        out_hbm.at[c, pl.ds(NS * ZROWS, ZTAIL)])


@functools.cache
def _make_sc_edge():
    mesh = plsc.VectorSubcoreMesh(
        core_axis_name="c", subcore_axis_name="s", num_cores=NC, num_subcores=NS
    )
    return pl.kernel(
        _sc_edge_body,
        mesh=mesh,
        out_type=jax.ShapeDtypeStruct((NC, N_NODES, D), jnp.float32),
        scratch_types=[
            pltpu.VMEM((2, NITC, EB), jnp.int32),
            pltpu.VMEM((2, NITC, EB), jnp.int32),
            pltpu.VMEM((2, EB, D), jnp.float32),
            pltpu.VMEM((2, EBP, D), jnp.int32),
            pltpu.VMEM_SHARED((N_NODES, D), jnp.float32),
            pltpu.SemaphoreType.DMA,
            pltpu.SemaphoreType.DMA,
            pltpu.SemaphoreType.DMA,
        ],
    )


def _sc_edge(h, ee, src_il, dst_il, zeros):
    return _make_sc_edge()(h, ee, src_il, dst_il, zeros)


_N_BLK = 1000
_N_GRID = N_NODES // _N_BLK


def _mlpA_body(x_ref, a0_ref, a1_ref, w_ref, b_ref, y_ref, st_ref):
    i = pl.program_id(0)
    h = x_ref[...] + a0_ref[...] + a1_ref[...]
    y = jnp.dot(h, w_ref[...], preferred_element_type=jnp.float32) + b_ref[...]
    y_ref[...] = y

    @pl.when(i == 0)
    def _():
        st_ref[...] = jnp.zeros_like(st_ref)

    st_ref[0:1, :] += jnp.sum(y, axis=0, keepdims=True)
    st_ref[1:2, :] += jnp.sum(y * y, axis=0, keepdims=True)


def _mlpA(x, a0, a1, w1, b1):
    return pl.pallas_call(
        _mlpA_body,
        grid=(_N_GRID,),
        in_specs=[
            pl.BlockSpec((_N_BLK, D), lambda i: (i, 0)),
            pl.BlockSpec((_N_BLK, D), lambda i: (i, 0)),
            pl.BlockSpec((_N_BLK, D), lambda i: (i, 0)),
            pl.BlockSpec((D, D), lambda i: (0, 0)),
            pl.BlockSpec((1, D), lambda i: (0, 0)),
        ],
        out_specs=[
            pl.BlockSpec((_N_BLK, D), lambda i: (i, 0)),
            pl.BlockSpec((8, D), lambda i: (0, 0)),
        ],
        out_shape=[
            jax.ShapeDtypeStruct((N_NODES, D), jnp.float32),
            jax.ShapeDtypeStruct((8, D), jnp.float32),
        ],
    )(x, a0, a1, w1, b1.reshape(1, D))


def _mlpB_body(y_ref, st_ref, g_ref, bb_ref, w2_ref, b2_ref, h_ref):
    inv_n = 1.0 / N_NODES
    mean = st_ref[0:1, :] * inv_n
    var = st_ref[1:2, :] * inv_n - mean * mean
    scale = lax.rsqrt(var + 1e-5) * g_ref[...]
    t = (y_ref[...] - mean) * scale + bb_ref[...]
    t = jnp.maximum(t, 0.0)
    h = jnp.dot(t, w2_ref[...], preferred_element_type=jnp.float32) + b2_ref[...]
    h_ref[...] = jnp.maximum(h, 0.0)


def _mlpB(y, st, g, bb, w2, b2):
    return pl.pallas_call(
        _mlpB_body,
        grid=(_N_GRID,),
        in_specs=[
            pl.BlockSpec((_N_BLK, D), lambda i: (i, 0)),
            pl.BlockSpec((8, D), lambda i: (0, 0)),
            pl.BlockSpec((1, D), lambda i: (0, 0)),
            pl.BlockSpec((1, D), lambda i: (0, 0)),
            pl.BlockSpec((D, D), lambda i: (0, 0)),
            pl.BlockSpec((1, D), lambda i: (0, 0)),
        ],
        out_specs=pl.BlockSpec((_N_BLK, D), lambda i: (i, 0)),
        out_shape=jax.ShapeDtypeStruct((N_NODES, D), jnp.float32),
    )(y, st, g.reshape(1, D), bb.reshape(1, D), w2, b2.reshape(1, D))


def _pool_body(bi_ref, h1_ref, h2_ref, h3_ref, h4_ref,
               g1_ref, g2_ref, g3_ref, g4_ref):
    i = pl.program_id(0)

    @pl.when(i == 0)
    def _():
        g1_ref[...] = jnp.zeros_like(g1_ref)
        g2_ref[...] = jnp.zeros_like(g2_ref)
        g3_ref[...] = jnp.zeros_like(g3_ref)
        g4_ref[...] = jnp.zeros_like(g4_ref)

    seg = bi_ref[0, :, :]
    gids = lax.broadcasted_iota(jnp.int32, (N_GRAPHS, _N_BLK), 0)
    onehot = (gids == seg).astype(jnp.float32)
    g1_ref[...] += jnp.dot(onehot, h1_ref[...], preferred_element_type=jnp.float32)
    g2_ref[...] += jnp.dot(onehot, h2_ref[...], preferred_element_type=jnp.float32)
    g3_ref[...] += jnp.dot(onehot, h3_ref[...], preferred_element_type=jnp.float32)
    g4_ref[...] += jnp.dot(onehot, h4_ref[...], preferred_element_type=jnp.float32)


def _pool(batch_index, h1, h2, h3, h4):
    bi = batch_index.reshape(_N_GRID, 1, _N_BLK)
    gspec = pl.BlockSpec((N_GRAPHS, D), lambda i: (0, 0))
    hspec = pl.BlockSpec((_N_BLK, D), lambda i: (i, 0))
    return pl.pallas_call(
        _pool_body,
        grid=(_N_GRID,),
        in_specs=[pl.BlockSpec((1, 1, _N_BLK), lambda i: (i, 0, 0)),
                  hspec, hspec, hspec, hspec],
        out_specs=[gspec, gspec, gspec, gspec],
        out_shape=[jax.ShapeDtypeStruct((N_GRAPHS, D), jnp.float32)] * 4,
    )(bi, h1, h2, h3, h4)


def _head_body(g_ref, w1_ref, b1_ref, w2_ref, b2_ref, o_ref):
    t = jnp.dot(g_ref[...], w1_ref[...], preferred_element_type=jnp.float32) + b1_ref[...]
    t = jnp.maximum(t, 0.0)
    o = jnp.dot(t, w2_ref[...], preferred_element_type=jnp.float32) + b2_ref[...]
    o_ref[...] = jnp.log1p(jnp.exp(-jnp.abs(o))) + jnp.maximum(o, 0.0)


def _head(g, w1, b1, w2, b2):
    return pl.pallas_call(
        _head_body,
        out_shape=jax.ShapeDtypeStruct((N_GRAPHS, N_OUT), jnp.float32),
    )(g, w1, b1.reshape(1, 4 * D), w2, b2.reshape(1, N_OUT))


def kernel(x, graph_level_feats, edge_attr, edge_index, batch_index,
           lin_e1_w, lin_e1_b, mlp1_w1, mlp1_b1, bn1_g, bn1_b, mlp1_w2, mlp1_b2,
           lin_e2_w, lin_e2_b, mlp2_w1, mlp2_b1, bn2_g, bn2_b, mlp2_w2, mlp2_b2,
           lin_e3_w, lin_e3_b, mlp3_w1, mlp3_b1, bn3_g, bn3_b, mlp3_w2, mlp3_b2,
           lin_e4_w, lin_e4_b, mlp4_w1, mlp4_b1, bn4_g, bn4_b, mlp4_w2, mlp4_b2,
           lin1_w, lin1_b, lin2_w, lin2_b):
    src = edge_index[0]
    dst = edge_index[1]
    zeros = jnp.zeros((ZROWS, D), jnp.float32)
    half = N_EDGES // 2

    def interleave(a):
        return jnp.stack([a[:half], a[half:]], axis=1).reshape(
            NC * NS, NCHUNK, NITC, EB)

    src_il = interleave(src)
    dst_il = interleave(dst)

    def conv(h, ee_w, ee_b, w1, b1, g, bb, w2, b2):
        ee = _edge_embed(edge_attr, ee_w, ee_b)
        agg = _sc_edge(h, ee, src_il, dst_il, zeros)
        y, st = _mlpA(h, agg[0], agg[1], w1, b1)
        return _mlpB(y, st, g, bb, w2, b2)

    h1 = conv(x, lin_e1_w, lin_e1_b, mlp1_w1, mlp1_b1, bn1_g, bn1_b, mlp1_w2, mlp1_b2)
    h2 = conv(h1, lin_e2_w, lin_e2_b, mlp2_w1, mlp2_b1, bn2_g, bn2_b, mlp2_w2, mlp2_b2)
    h3 = conv(h2, lin_e3_w, lin_e3_b, mlp3_w1, mlp3_b1, bn3_g, bn3_b, mlp3_w2, mlp3_b2)
    h4 = conv(h2, lin_e4_w, lin_e4_b, mlp4_w1, mlp4_b1, bn4_g, bn4_b, mlp4_w2, mlp4_b2)

    g1, g2, g3, g4 = _pool(batch_index, h1, h2, h3, h4)
    g = jnp.concatenate((g1, g2, g3, g4), axis=1)
    return _head(g, lin1_w, lin1_b, lin2_w, lin2_b)


# speedup vs baseline: 1.3581x; 1.1919x over previous
import functools

import jax
import jax.numpy as jnp
import numpy as np
from jax import lax
from jax.experimental import pallas as pl
from jax.experimental.pallas import tpu as pltpu
from jax.experimental.pallas import tpu_sc as plsc

N_NODES = 10000
N_EDGES = 320000
D = 128
D_EDGE = 16
N_GRAPHS = 64
N_OUT = 64

NC = 2
NS = 16
EBP = 40
EB = 2 * EBP
EPW = N_EDGES // (NC * NS)
NIT = EPW // EB
NITC = 5
NCHUNK = NIT // NITC
EPWP = EPW // 2
ZROWS = 624
ZTAIL = N_NODES - NS * ZROWS


def _pack_bf16_pairs(ylo, yhi):
    def rne(v):
        u = lax.bitcast_convert_type(v, jnp.uint32)
        return (u + jnp.uint32(0x7FFF) + ((u >> 16) & jnp.uint32(1))) >> 16

    packed = rne(ylo) | (rne(yhi) << 16)
    return lax.bitcast_convert_type(packed, jnp.int32)


_EE_BLK = 2000


_EE_GRID = (N_EDGES // 2) // _EE_BLK


def _ee_body(ea_lo_ref, ea_hi_ref, w_ref, b_ref, out_ref):
    lo = (jnp.dot(ea_lo_ref[...], w_ref[...], preferred_element_type=jnp.float32)
          + b_ref[...])
    hi = (jnp.dot(ea_hi_ref[...], w_ref[...], preferred_element_type=jnp.float32)
          + b_ref[...])
    out_ref[...] = _pack_bf16_pairs(lo, hi)


def _edge_embed(edge_attr, w, b):
    return pl.pallas_call(
        _ee_body,
        grid=(_EE_GRID,),
        in_specs=[
            pl.BlockSpec((_EE_BLK, D_EDGE), lambda i: (i, 0)),
            pl.BlockSpec((_EE_BLK, D_EDGE), lambda i: (i + _EE_GRID, 0)),
            pl.BlockSpec((D_EDGE, D), lambda i: (0, 0)),
            pl.BlockSpec((1, D), lambda i: (0, 0)),
        ],
        out_specs=pl.BlockSpec((_EE_BLK, D), lambda i: (i, 0)),
        out_shape=jax.ShapeDtypeStruct((N_EDGES // 2, D), jnp.int32),
    )(edge_attr, edge_attr, w, b.reshape(1, D))


def _sc_edge_body(x_hbm, ee_hbm, src_hbm, dst_hbm, zeros_hbm, out_hbm,
                  src_v, dst_v, xg, eeb, aggr, e_sem, g_sem, s_sem):
    c = lax.axis_index("c")
    s = lax.axis_index("s")

    pltpu.sync_copy(zeros_hbm, aggr.at[pl.ds(s * ZROWS, ZROWS)])

    @pl.when(s == NS - 1)
    def _():
        pltpu.sync_copy(zeros_hbm.at[pl.ds(0, ZTAIL)],
                        aggr.at[pl.ds(NS * ZROWS, ZTAIL)])

    w = c * NS + s

    def load_idx_chunk(ch):
        b = lax.rem(ch, 2)
        pltpu.sync_copy(src_hbm.at[w, ch], src_v.at[b])
        pltpu.sync_copy(dst_hbm.at[w, ch], dst_v.at[b])

    load_idx_chunk(0)
    plsc.subcore_barrier()

    pbase = w * EPWP

    def src_row(i):
        return src_v.at[lax.rem(i // NITC, 2), lax.rem(i, NITC)]

    def dst_row(i):
        return dst_v.at[lax.rem(i // NITC, 2), lax.rem(i, NITC)]

    def start_ee(i, b):
        pltpu.async_copy(
            ee_hbm.at[pl.ds(pbase + i * EBP, EBP)], eeb.at[b], e_sem)

    def wait_ee(i, b):
        pltpu.make_async_copy(
            ee_hbm.at[pl.ds(pbase + i * EBP, EBP)], eeb.at[b], e_sem).wait()

    def start_gather(i, b):
        pltpu.async_copy(x_hbm.at[src_row(i)], xg.at[b], g_sem)

    def wait_gather(i, b):
        pltpu.make_async_copy(x_hbm.at[src_row(i)], xg.at[b], g_sem).wait()

    def start_scatter(i, b):
        pltpu.async_copy(xg.at[b], aggr.at[dst_row(i)], s_sem, add=True)

    def wait_scatter(i, b):
        pltpu.make_async_copy(xg.at[b], aggr.at[dst_row(i)], s_sem).wait()

    start_ee(0, 0)
    start_gather(0, 0)
    start_gather(1, 1)

    hi_mask = jnp.int32(-65536)

    def edge_iter(i, carry):
        ebuf = lax.rem(i, 2)
        gbuf = lax.rem(i, 3)

        @pl.when(jnp.logical_and(i + 2 < NIT, lax.rem(i + 2, NITC) == 0))
        def _():
            load_idx_chunk((i + 2) // NITC)

        @pl.when(i >= 1)
        def _():
            wait_scatter(i - 1, lax.rem(i - 1, 3))

        @pl.when(i + 1 < NIT)
        def _():
            start_ee(i + 1, 1 - ebuf)

        @pl.when(i + 2 < NIT)
        def _():
            start_gather(i + 2, lax.rem(i + 2, 3))

        wait_ee(i, ebuf)
        wait_gather(i, gbuf)

        @plsc.parallel_loop(0, EBP, unroll=4)
        def _(t):
            for j in range(D // 16):
                cs = pl.ds(j * 16, 16)
                v = eeb[ebuf, t, cs]
                lo = lax.bitcast_convert_type(v << 16, jnp.float32)
                hi = lax.bitcast_convert_type(v & hi_mask, jnp.float32)
                xg[gbuf, 2 * t, cs] = jnp.maximum(xg[gbuf, 2 * t, cs] + lo, 0.0)
                xg[gbuf, 2 * t + 1, cs] = jnp.maximum(
                    xg[gbuf, 2 * t + 1, cs] + hi, 0.0)

        start_scatter(i, gbuf)
        return carry

    lax.fori_loop(0, NIT, edge_iter, 0)
    wait_scatter(NIT - 1, lax.rem(NIT - 1, 3))
    plsc.subcore_barrier()

    pltpu.sync_copy(
        aggr.at[pl.ds(s * ZROWS, ZROWS)],
        out_hbm.at[c, pl.ds(s * ZROWS, ZROWS)],
    )

    @pl.when(s == NS - 1)
    def _():
        pltpu.sync_copy(aggr.at[pl.ds(NS * ZROWS, ZTAIL)],
                        out_hbm.at[c, pl.ds(NS * ZROWS, ZTAIL)])


@functools.cache
def _make_sc_edge():
    mesh = plsc.VectorSubcoreMesh(
        core_axis_name="c", subcore_axis_name="s", num_cores=NC, num_subcores=NS
    )
    return pl.kernel(
        _sc_edge_body,
        mesh=mesh,
        out_type=jax.ShapeDtypeStruct((NC, N_NODES, D), jnp.float32),
        scratch_types=[
            pltpu.VMEM((2, NITC, EB), jnp.int32),
            pltpu.VMEM((2, NITC, EB), jnp.int32),
            pltpu.VMEM((3, EB, D), jnp.float32),
            pltpu.VMEM((2, EBP, D), jnp.int32),
            pltpu.VMEM_SHARED((N_NODES, D), jnp.float32),
            pltpu.SemaphoreType.DMA,
            pltpu.SemaphoreType.DMA,
            pltpu.SemaphoreType.DMA,
        ],
    )


def _sc_edge(h, ee, src_il, dst_il, zeros):
    return _make_sc_edge()(h, ee, src_il, dst_il, zeros)


_N_BLK = 1000
_N_GRID = N_NODES // _N_BLK


def _mlpA_body(x_ref, a0_ref, a1_ref, w_ref, b_ref, y_ref, st_ref):
    i = pl.program_id(0)
    h = x_ref[...] + a0_ref[...] + a1_ref[...]
    y = jnp.dot(h, w_ref[...], preferred_element_type=jnp.float32) + b_ref[...]
    y_ref[...] = y

    @pl.when(i == 0)
    def _():
        st_ref[...] = jnp.zeros_like(st_ref)

    st_ref[0:1, :] += jnp.sum(y, axis=0, keepdims=True)
    st_ref[1:2, :] += jnp.sum(y * y, axis=0, keepdims=True)


def _mlpA(x, a0, a1, w1, b1):
    return pl.pallas_call(
        _mlpA_body,
        grid=(_N_GRID,),
        in_specs=[
            pl.BlockSpec((_N_BLK, D), lambda i: (i, 0)),
            pl.BlockSpec((_N_BLK, D), lambda i: (i, 0)),
            pl.BlockSpec((_N_BLK, D), lambda i: (i, 0)),
            pl.BlockSpec((D, D), lambda i: (0, 0)),
            pl.BlockSpec((1, D), lambda i: (0, 0)),
        ],
        out_specs=[
            pl.BlockSpec((_N_BLK, D), lambda i: (i, 0)),
            pl.BlockSpec((8, D), lambda i: (0, 0)),
        ],
        out_shape=[
            jax.ShapeDtypeStruct((N_NODES, D), jnp.float32),
            jax.ShapeDtypeStruct((8, D), jnp.float32),
        ],
    )(x, a0, a1, w1, b1.reshape(1, D))


def _mlpB_body(y_ref, st_ref, g_ref, bb_ref, w2_ref, b2_ref, h_ref):
    inv_n = 1.0 / N_NODES
    mean = st_ref[0:1, :] * inv_n
    var = st_ref[1:2, :] * inv_n - mean * mean
    scale = lax.rsqrt(var + 1e-5) * g_ref[...]
    t = (y_ref[...] - mean) * scale + bb_ref[...]
    t = jnp.maximum(t, 0.0)
    h = jnp.dot(t, w2_ref[...], preferred_element_type=jnp.float32) + b2_ref[...]
    h_ref[...] = jnp.maximum(h, 0.0)


def _mlpB(y, st, g, bb, w2, b2):
    return pl.pallas_call(
        _mlpB_body,
        grid=(_N_GRID,),
        in_specs=[
            pl.BlockSpec((_N_BLK, D), lambda i: (i, 0)),
            pl.BlockSpec((8, D), lambda i: (0, 0)),
            pl.BlockSpec((1, D), lambda i: (0, 0)),
            pl.BlockSpec((1, D), lambda i: (0, 0)),
            pl.BlockSpec((D, D), lambda i: (0, 0)),
            pl.BlockSpec((1, D), lambda i: (0, 0)),
        ],
        out_specs=pl.BlockSpec((_N_BLK, D), lambda i: (i, 0)),
        out_shape=jax.ShapeDtypeStruct((N_NODES, D), jnp.float32),
    )(y, st, g.reshape(1, D), bb.reshape(1, D), w2, b2.reshape(1, D))


def _pool_body(bi_ref, h1_ref, h2_ref, h3_ref, h4_ref,
               g1_ref, g2_ref, g3_ref, g4_ref):
    i = pl.program_id(0)

    @pl.when(i == 0)
    def _():
        g1_ref[...] = jnp.zeros_like(g1_ref)
        g2_ref[...] = jnp.zeros_like(g2_ref)
        g3_ref[...] = jnp.zeros_like(g3_ref)
        g4_ref[...] = jnp.zeros_like(g4_ref)

    seg = bi_ref[0, :, :]
    gids = lax.broadcasted_iota(jnp.int32, (N_GRAPHS, _N_BLK), 0)
    onehot = (gids == seg).astype(jnp.float32)
    g1_ref[...] += jnp.dot(onehot, h1_ref[...], preferred_element_type=jnp.float32)
    g2_ref[...] += jnp.dot(onehot, h2_ref[...], preferred_element_type=jnp.float32)
    g3_ref[...] += jnp.dot(onehot, h3_ref[...], preferred_element_type=jnp.float32)
    g4_ref[...] += jnp.dot(onehot, h4_ref[...], preferred_element_type=jnp.float32)


def _pool(batch_index, h1, h2, h3, h4):
    bi = batch_index.reshape(_N_GRID, 1, _N_BLK)
    gspec = pl.BlockSpec((N_GRAPHS, D), lambda i: (0, 0))
    hspec = pl.BlockSpec((_N_BLK, D), lambda i: (i, 0))
    return pl.pallas_call(
        _pool_body,
        grid=(_N_GRID,),
        in_specs=[pl.BlockSpec((1, 1, _N_BLK), lambda i: (i, 0, 0)),
                  hspec, hspec, hspec, hspec],
        out_specs=[gspec, gspec, gspec, gspec],
        out_shape=[jax.ShapeDtypeStruct((N_GRAPHS, D), jnp.float32)] * 4,
    )(bi, h1, h2, h3, h4)


def _head_body(g_ref, w1_ref, b1_ref, w2_ref, b2_ref, o_ref):
    t = jnp.dot(g_ref[...], w1_ref[...], preferred_element_type=jnp.float32) + b1_ref[...]
    t = jnp.maximum(t, 0.0)
    o = jnp.dot(t, w2_ref[...], preferred_element_type=jnp.float32) + b2_ref[...]
    o_ref[...] = jnp.log1p(jnp.exp(-jnp.abs(o))) + jnp.maximum(o, 0.0)


def _head(g, w1, b1, w2, b2):
    return pl.pallas_call(
        _head_body,
        out_shape=jax.ShapeDtypeStruct((N_GRAPHS, N_OUT), jnp.float32),
    )(g, w1, b1.reshape(1, 4 * D), w2, b2.reshape(1, N_OUT))


def kernel(x, graph_level_feats, edge_attr, edge_index, batch_index,
           lin_e1_w, lin_e1_b, mlp1_w1, mlp1_b1, bn1_g, bn1_b, mlp1_w2, mlp1_b2,
           lin_e2_w, lin_e2_b, mlp2_w1, mlp2_b1, bn2_g, bn2_b, mlp2_w2, mlp2_b2,
           lin_e3_w, lin_e3_b, mlp3_w1, mlp3_b1, bn3_g, bn3_b, mlp3_w2, mlp3_b2,
           lin_e4_w, lin_e4_b, mlp4_w1, mlp4_b1, bn4_g, bn4_b, mlp4_w2, mlp4_b2,
           lin1_w, lin1_b, lin2_w, lin2_b):
    src = edge_index[0]
    dst = edge_index[1]
    zeros = jnp.zeros((ZROWS, D), jnp.float32)
    half = N_EDGES // 2

    def interleave(a):
        return jnp.stack([a[:half], a[half:]], axis=1).reshape(
            NC * NS, NCHUNK, NITC, EB)

    src_il = interleave(src)
    dst_il = interleave(dst)

    ee1 = _edge_embed(edge_attr, lin_e1_w, lin_e1_b)
    ee2 = _edge_embed(edge_attr, lin_e2_w, lin_e2_b)
    ee3 = _edge_embed(edge_attr, lin_e3_w, lin_e3_b)
    ee4 = _edge_embed(edge_attr, lin_e4_w, lin_e4_b)

    def conv(h, ee, w1, b1, g, bb, w2, b2):
        agg = _sc_edge(h, ee, src_il, dst_il, zeros)
        y, st = _mlpA(h, agg[0], agg[1], w1, b1)
        return _mlpB(y, st, g, bb, w2, b2)

    h1 = conv(x, ee1, mlp1_w1, mlp1_b1, bn1_g, bn1_b, mlp1_w2, mlp1_b2)
    h2 = conv(h1, ee2, mlp2_w1, mlp2_b1, bn2_g, bn2_b, mlp2_w2, mlp2_b2)
    h3 = conv(h2, ee3, mlp3_w1, mlp3_b1, bn3_g, bn3_b, mlp3_w2, mlp3_b2)
    h4 = conv(h2, ee4, mlp4_w1, mlp4_b1, bn4_g, bn4_b, mlp4_w2, mlp4_b2)

    g1, g2, g3, g4 = _pool(batch_index, h1, h2, h3, h4)
    g = jnp.concatenate((g1, g2, g3, g4), axis=1)
    return _head(g, lin1_w, lin1_b, lin2_w, lin2_b)
